# baseline (speedup 1.0000x reference)
import jax
import jax.numpy as jnp
from jax.experimental import pallas as pl
from jax.experimental.pallas import tpu as pltpu

_B, _C, _H, _W = 64, 768, 16, 16
_HW = _H * _W
_E, _TOPK = 8, 2
_PAD = 128


def _probe_kernel(x_hbm, b_ref, idx_ref, wgt_ref):
    lanes = jax.lax.broadcasted_iota(jnp.int32, (_B, _PAD), 1)
    wgt_ref[...] = jnp.broadcast_to(b_ref[0, 0], (_B, _PAD)) * 0.0
    idx_ref[...] = lanes * 0


def kernel(x, fc_w, fc_b):
    xr = x.reshape(-1)
    br = fc_b.reshape(1, _E)
    idx_pad, wgt_pad = pl.pallas_call(
        _probe_kernel,
        in_specs=[
            pl.BlockSpec(memory_space=pltpu.MemorySpace.HBM),
            pl.BlockSpec((1, _E), lambda: (0, 0)),
        ],
        out_specs=[
            pl.BlockSpec((_B, _PAD), lambda: (0, 0)),
            pl.BlockSpec((_B, _PAD), lambda: (0, 0)),
        ],
        out_shape=[
            jax.ShapeDtypeStruct((_B, _PAD), jnp.int32),
            jax.ShapeDtypeStruct((_B, _PAD), jnp.float32),
        ],
    )(xr, br)
    return idx_pad[:, :_TOPK], wgt_pad[:, :_TOPK]


# speedup vs baseline: 17.6582x; 17.6582x over previous
import jax
import jax.numpy as jnp
from jax.experimental import pallas as pl
from jax.experimental.pallas import tpu as pltpu

_B, _C, _H, _W = 64, 768, 16, 16
_HW = _H * _W
_E, _TOPK = 8, 2
_BB = 4
_NCHUNK = _B // _BB
_NBUF = 8
_PAD = 128


def _router_kernel(x_hbm, w_ref, b_ref, idx_ref, wgt_ref, buf, sc, sem):
    def start(chunk, slot):
        pltpu.make_async_copy(
            x_hbm.at[pl.ds(chunk * _BB, _BB)], buf.at[slot], sem.at[slot]
        ).start()

    for k in range(_NBUF):
        start(k, k)

    for chunk in range(_NCHUNK):
        slot = chunk % _NBUF
        pltpu.make_async_copy(
            x_hbm.at[pl.ds(chunk * _BB, _BB)], buf.at[slot], sem.at[slot]
        ).wait()
        xb = buf[slot]
        pooled = jnp.mean(xb, axis=1)
        sc[pl.ds(chunk * _BB, _BB), :] = jax.lax.dot_general(
            pooled, w_ref[...],
            dimension_numbers=(((1,), (1,)), ((), ())),
            preferred_element_type=jnp.float32) + b_ref[...]
        nxt = chunk + _NBUF
        if nxt < _NCHUNK:
            start(nxt, slot)

    scores = sc[...]
    m = jnp.max(scores, axis=1, keepdims=True)
    ex = jnp.exp(scores - m)
    probs = ex / jnp.sum(ex, axis=1, keepdims=True)

    cols = jax.lax.broadcasted_iota(jnp.int32, (_B, _E), 1)
    p1 = jnp.max(probs, axis=1, keepdims=True)
    i1 = jnp.argmax(probs, axis=1)[:, None]
    masked = jnp.where(cols == i1, -jnp.inf, probs)
    p2 = jnp.max(masked, axis=1, keepdims=True)
    i2 = jnp.argmax(masked, axis=1)[:, None]
    s = p1 + p2

    lanes = jax.lax.broadcasted_iota(jnp.int32, (_B, _PAD), 1)
    wgt_ref[...] = jnp.where(lanes == 0, p1 / s,
                             jnp.where(lanes == 1, p2 / s, 0.0))
    idx_ref[...] = jnp.where(lanes == 0, i1,
                             jnp.where(lanes == 1, i2, 0))


def kernel(x, fc_w, fc_b):
    xr = jnp.transpose(x, (0, 2, 3, 1)).reshape(_B, _HW, _C)
    br = fc_b.reshape(1, _E)
    idx_pad, wgt_pad = pl.pallas_call(
        _router_kernel,
        in_specs=[
            pl.BlockSpec(memory_space=pltpu.MemorySpace.HBM),
            pl.BlockSpec((_E, _C), lambda: (0, 0)),
            pl.BlockSpec((1, _E), lambda: (0, 0)),
        ],
        out_specs=[
            pl.BlockSpec((_B, _PAD), lambda: (0, 0)),
            pl.BlockSpec((_B, _PAD), lambda: (0, 0)),
        ],
        out_shape=[
            jax.ShapeDtypeStruct((_B, _PAD), jnp.int32),
            jax.ShapeDtypeStruct((_B, _PAD), jnp.float32),
        ],
        scratch_shapes=[
            pltpu.VMEM((_NBUF, _BB, _HW, _C), jnp.float32),
            pltpu.VMEM((_B, _E), jnp.float32),
            pltpu.SemaphoreType.DMA((_NBUF,)),
        ],
    )(xr, fc_w, br)
    return idx_pad[:, :_TOPK], wgt_pad[:, :_TOPK]


# speedup vs baseline: 18.1371x; 1.0271x over previous
import jax
import jax.numpy as jnp
from jax.experimental import pallas as pl
from jax.experimental.pallas import tpu as pltpu

_B, _C, _H, _W = 64, 768, 16, 16
_HW = _H * _W
_E, _TOPK = 8, 2
_BB = 4
_NCHUNK = _B // _BB
_NBUF = 8


def _router_kernel(x_hbm, w_ref, b_ref, idx_ref, wgt_ref, buf, pooled_sc, sem):
    def start(chunk, slot):
        pltpu.make_async_copy(
            x_hbm.at[pl.ds(chunk * _BB, _BB)], buf.at[slot], sem.at[slot]
        ).start()

    for k in range(_NBUF):
        start(k, k)

    for chunk in range(_NCHUNK):
        slot = chunk % _NBUF
        pltpu.make_async_copy(
            x_hbm.at[pl.ds(chunk * _BB, _BB)], buf.at[slot], sem.at[slot]
        ).wait()
        xb = buf[slot]
        pooled_sc[pl.ds(chunk * _BB, _BB), :] = jnp.mean(xb, axis=1)
        nxt = chunk + _NBUF
        if nxt < _NCHUNK:
            start(nxt, slot)

    st = jax.lax.dot_general(
        w_ref[...], pooled_sc[...],
        dimension_numbers=(((1,), (1,)), ((), ())),
        preferred_element_type=jnp.float32)
    b_col = jax.lax.dot_general(
        jnp.eye(_E, dtype=jnp.float32), b_ref[...],
        dimension_numbers=(((1,), (1,)), ((), ())),
        preferred_element_type=jnp.float32)
    st = st + b_col

    m = jnp.max(st, axis=0, keepdims=True)
    ex = jnp.exp(st - m)
    probs = ex / jnp.sum(ex, axis=0, keepdims=True)

    rows = jax.lax.broadcasted_iota(jnp.int32, (_E, _B), 0)
    p1 = jnp.max(probs, axis=0, keepdims=True)
    i1 = jnp.min(jnp.where(probs == p1, rows, _E), axis=0, keepdims=True)
    masked = jnp.where(rows == i1, -jnp.inf, probs)
    p2 = jnp.max(masked, axis=0, keepdims=True)
    i2 = jnp.min(jnp.where(masked == p2, rows, _E), axis=0, keepdims=True)
    s = p1 + p2

    wgt_ref[0:1, 0:_B] = p1 / s
    wgt_ref[1:2, 0:_B] = p2 / s
    idx_ref[0:1, 0:_B] = i1
    idx_ref[1:2, 0:_B] = i2


def kernel(x, fc_w, fc_b):
    xr = jnp.transpose(x, (0, 2, 3, 1)).reshape(_B, _HW, _C)
    br = fc_b.reshape(1, _E)
    idx_t, wgt_t = pl.pallas_call(
        _router_kernel,
        in_specs=[
            pl.BlockSpec(memory_space=pltpu.MemorySpace.HBM),
            pl.BlockSpec((_E, _C), lambda: (0, 0)),
            pl.BlockSpec((1, _E), lambda: (0, 0)),
        ],
        out_specs=[
            pl.BlockSpec((8, 128), lambda: (0, 0)),
            pl.BlockSpec((8, 128), lambda: (0, 0)),
        ],
        out_shape=[
            jax.ShapeDtypeStruct((8, 128), jnp.int32),
            jax.ShapeDtypeStruct((8, 128), jnp.float32),
        ],
        scratch_shapes=[
            pltpu.VMEM((_NBUF, _BB, _HW, _C), jnp.float32),
            pltpu.VMEM((_B, _C), jnp.float32),
            pltpu.SemaphoreType.DMA((_NBUF,)),
        ],
    )(xr, fc_w, br)
    top_idx = jnp.stack([idx_t[0, :_B], idx_t[1, :_B]], axis=1)
    top_wgt = jnp.stack([wgt_t[0, :_B], wgt_t[1, :_B]], axis=1)
    return top_idx, top_wgt


# speedup vs baseline: 19.2495x; 1.0613x over previous
import jax
import jax.numpy as jnp
from jax.experimental import pallas as pl
from jax.experimental.pallas import tpu as pltpu

_B, _C, _H, _W = 64, 768, 16, 16
_HW = _H * _W
_E, _TOPK = 8, 2
_BB = 4
_NCHUNK = _B // _BB
_NBUF = 8


def _router_kernel(x_hbm, w_ref, b_ref, out_ref, buf, pooled_sc, sem):
    def start(chunk, slot):
        pltpu.make_async_copy(
            x_hbm.at[pl.ds(chunk * _BB, _BB)], buf.at[slot], sem.at[slot]
        ).start()

    for k in range(_NBUF):
        start(k, k)

    for chunk in range(_NCHUNK):
        slot = chunk % _NBUF
        pltpu.make_async_copy(
            x_hbm.at[pl.ds(chunk * _BB, _BB)], buf.at[slot], sem.at[slot]
        ).wait()
        xb = buf[slot]
        pooled_sc[pl.ds(chunk * _BB, _BB), :] = jnp.mean(xb, axis=1)
        nxt = chunk + _NBUF
        if nxt < _NCHUNK:
            start(nxt, slot)

    st = jax.lax.dot_general(
        w_ref[...], pooled_sc[...],
        dimension_numbers=(((1,), (1,)), ((), ())),
        preferred_element_type=jnp.float32)
    b_col = jax.lax.dot_general(
        jnp.eye(_E, dtype=jnp.float32), b_ref[...],
        dimension_numbers=(((1,), (1,)), ((), ())),
        preferred_element_type=jnp.float32)
    st = st + b_col

    m = jnp.max(st, axis=0, keepdims=True)
    ex = jnp.exp(st - m)
    probs = ex / jnp.sum(ex, axis=0, keepdims=True)

    rows = jax.lax.broadcasted_iota(jnp.int32, (_E, _B), 0)
    p1 = jnp.max(probs, axis=0, keepdims=True)
    i1 = jnp.min(jnp.where(probs == p1, rows, _E), axis=0, keepdims=True)
    masked = jnp.where(rows == i1, -jnp.inf, probs)
    p2 = jnp.max(masked, axis=0, keepdims=True)
    i2 = jnp.min(jnp.where(masked == p2, rows, _E), axis=0, keepdims=True)
    s = p1 + p2

    out_ref[0:1, 0:_B] = p1 / s
    out_ref[1:2, 0:_B] = p2 / s
    out_ref[2:3, 0:_B] = i1.astype(jnp.float32)
    out_ref[3:4, 0:_B] = i2.astype(jnp.float32)


def kernel(x, fc_w, fc_b):
    xr = jnp.transpose(x, (0, 2, 3, 1)).reshape(_B, _HW, _C)
    br = fc_b.reshape(1, _E)
    out_t = pl.pallas_call(
        _router_kernel,
        in_specs=[
            pl.BlockSpec(memory_space=pltpu.MemorySpace.HBM),
            pl.BlockSpec((_E, _C), lambda: (0, 0)),
            pl.BlockSpec((1, _E), lambda: (0, 0)),
        ],
        out_specs=pl.BlockSpec((8, 128), lambda: (0, 0)),
        out_shape=jax.ShapeDtypeStruct((8, 128), jnp.float32),
        scratch_shapes=[
            pltpu.VMEM((_NBUF, _BB, _HW, _C), jnp.float32),
            pltpu.VMEM((_B, _C), jnp.float32),
            pltpu.SemaphoreType.DMA((_NBUF,)),
        ],
    )(xr, fc_w, br)
    top_wgt = jnp.stack([out_t[0, :_B], out_t[1, :_B]], axis=1)
    top_idx = jnp.stack([out_t[2, :_B], out_t[3, :_B]], axis=1).astype(jnp.int32)
    return top_idx, top_wgt
